# baseline (speedup 1.0000x reference)
import functools

import jax
import jax.numpy as jnp
from jax.experimental import pallas as pl


def _body(x_ref, m_ref, o_ref, *, hw, r):
    m = m_ref[0]
    is_fg = m >= 0.5
    fg = is_fg.astype(jnp.float32)
    n_fg = jnp.sum(fg)
    n_bg = hw - n_fg

    x = x_ref[...]
    xsq = x * x
    xm = jnp.where(is_fg, x, 0.0)
    xmsq = jnp.where(is_fg, xsq, 0.0)
    s_all = jnp.sum(x, axis=(1, 2))
    s_fg = jnp.sum(xm, axis=(1, 2))
    q_all = jnp.sum(xsq, axis=(1, 2))
    q_fg = jnp.sum(xmsq, axis=(1, 2))

    mu_fg = s_fg / n_fg
    mu_bg = (s_all - s_fg) / n_bg
    var_fg = (q_fg - n_fg * mu_fg * mu_fg) / (n_fg - 1.0)
    var_bg = ((q_all - q_fg) - n_bg * mu_bg * mu_bg) / (n_bg - 1.0)
    scale = jnp.sqrt(var_fg) / (jnp.sqrt(var_bg) + 1e-8)
    shift = mu_fg - scale * mu_bg

    y = x * scale[:, None, None] + shift[:, None, None]
    o_ref[...] = jnp.where(is_fg, x, y)


def kernel(x, mask):
    b, c, h, w = x.shape
    hw = h * w
    s = hw // 128
    x3 = x.reshape(b * c, s, 128)
    m3 = mask.reshape(b, s, 128)

    r = 8 if c % 8 == 0 else 1
    grid = (b * c) // r
    rows_per_b = c // r

    out = pl.pallas_call(
        functools.partial(_body, hw=float(hw), r=r),
        grid=(grid,),
        in_specs=[
            pl.BlockSpec((r, s, 128), lambda i: (i, 0, 0)),
            pl.BlockSpec((1, s, 128), lambda i: (i // rows_per_b, 0, 0)),
        ],
        out_specs=pl.BlockSpec((r, s, 128), lambda i: (i, 0, 0)),
        out_shape=jax.ShapeDtypeStruct((b * c, s, 128), x.dtype),
    )(x3, m3)
    return out.reshape(b, c, h, w)
